# baseline (speedup 1.0000x reference)
import jax
import jax.numpy as jnp
from jax import lax
from jax.experimental import pallas as pl

_F32 = jnp.float32


def _row_block(n, target):
    for br in (target, 200, 80, 40, 8):
        if br <= n and n % br == 0:
            return br
    return n


def _full(shape):
    return pl.BlockSpec(shape, lambda i: (0,) * len(shape))


def _rows(br, ncols):
    return pl.BlockSpec((br, ncols), lambda i: (i, 0))


def _pre_body(x_ref, w1s_ref, w1f_ref, os_ref, of_ref):
    xb = x_ref[...]
    os_ref[...] = jnp.dot(xb, w1s_ref[...], preferred_element_type=_F32)
    of_ref[...] = jnp.dot(xb, w1f_ref[...], preferred_element_type=_F32)


def _passA_body(as_ref, af_ref, us_ref, uf_ref, b1s_ref, b1f_ref,
                w2s_ref, w2f_ref, ts_ref, tf_ref,
                ms_ref, mf_ref, ss_ref, sf_ref):
    a_s = as_ref[...]
    a_f = af_ref[...]
    hs = jnp.maximum(
        jnp.dot(a_s, us_ref[...], preferred_element_type=_F32)
        + b1s_ref[...], 0.0)
    ts_ref[...] = jnp.dot(hs, w2s_ref[...], preferred_element_type=_F32)
    hf = jnp.maximum(
        jnp.dot(a_f, uf_ref[...], preferred_element_type=_F32)
        + b1f_ref[...], 0.0)
    tf_ref[...] = jnp.dot(hf, w2f_ref[...], preferred_element_type=_F32)
    idx = lax.broadcasted_iota(jnp.int32, a_s.shape, 1)
    in_row = idx < a_s.shape[1]
    mn_s = jnp.min(jnp.where((a_s > 0.0) & in_row, a_s, 1.0), axis=1,
                   keepdims=True)
    mn_f = jnp.min(jnp.where((a_f > 0.0) & in_row, a_f, 1.0), axis=1,
                   keepdims=True)
    ss_ref[...] = mn_s
    sf_ref[...] = mn_f
    ms_ref[...] = ((a_s > 0.0).astype(_F32)
                   + (a_s > 1.5 * mn_s).astype(_F32)).astype(jnp.int8)
    mf_ref[...] = ((a_f > 0.0).astype(_F32)
                   + (a_f > 1.5 * mn_f).astype(_F32)).astype(jnp.int8)


def _passB_body(ms_ref, mf_ref, ss_ref, sf_ref, ts_ref, tf_ref,
                b2s_ref, b2f_ref, attW_ref, attb_ref, attq_ref,
                mlpW_ref, mlpb_ref, decW1_ref,
                hm_ref, att_ref, hn_ref, hd_ref):
    a_s_rec = ss_ref[...] * ms_ref[...].astype(_F32)
    a_f_rec = sf_ref[...] * mf_ref[...].astype(_F32)
    g_s = (jnp.dot(a_s_rec, ts_ref[...], preferred_element_type=_F32)
           + b2s_ref[...])
    g_f = (jnp.dot(a_f_rec, tf_ref[...], preferred_element_type=_F32)
           + b2f_ref[...])
    w_s = jnp.tanh(jnp.dot(g_s, attW_ref[...], preferred_element_type=_F32)
                   + attb_ref[...])
    w_f = jnp.tanh(jnp.dot(g_f, attW_ref[...], preferred_element_type=_F32)
                   + attb_ref[...])
    sc_s = jnp.dot(w_s, attq_ref[...], preferred_element_type=_F32)
    sc_f = jnp.dot(w_f, attq_ref[...], preferred_element_type=_F32)
    m = jnp.maximum(sc_s, sc_f)
    es = jnp.exp(sc_s - m)
    ef = jnp.exp(sc_f - m)
    den = es + ef
    a_s = es / den
    a_f = ef / den
    h = a_s * g_s + a_f * g_f
    hm = jnp.dot(h, mlpW_ref[...], preferred_element_type=_F32) + mlpb_ref[...]
    hm_ref[...] = hm
    att_ref[...] = jnp.concatenate([a_s, a_f], axis=1)
    nrm = jnp.sqrt(jnp.sum(hm * hm, axis=1, keepdims=True))
    hn_ref[...] = hm / (nrm + 1e-8)
    hd_ref[...] = jnp.dot(hm, decW1_ref[...], preferred_element_type=_F32)


def _passC_body(ms_ref, ss_ref, hd_ref, hnb_ref, hn_ref, db1_ref,
                wpi_ref, bpi_ref, wdisp_ref, bdisp_ref, wmean_ref, bmean_ref,
                pi_ref, disp_ref, mean_ref, recon_ref):
    a_s_rec = ss_ref[...] * ms_ref[...].astype(_F32)
    h1 = jnp.maximum(
        jnp.dot(a_s_rec, hd_ref[...], preferred_element_type=_F32)
        + db1_ref[...], 0.0)
    zpi = jnp.dot(h1, wpi_ref[...], preferred_element_type=_F32) + bpi_ref[...]
    pi_ref[...] = 1.0 / (1.0 + jnp.exp(-zpi))
    zd = jnp.dot(h1, wdisp_ref[...], preferred_element_type=_F32) + bdisp_ref[...]
    sp = jnp.maximum(zd, 0.0) + jnp.log1p(jnp.exp(-jnp.abs(zd)))
    disp_ref[...] = jnp.clip(sp, 1e-4, 1e4)
    zm = jnp.dot(h1, wmean_ref[...], preferred_element_type=_F32) + bmean_ref[...]
    mean_ref[...] = jnp.clip(jnp.exp(zm), 1e-5, 1e6)
    recon_ref[...] = lax.dot_general(
        hnb_ref[...], hn_ref[...], (((1,), (1,)), ((), ())),
        preferred_element_type=_F32)


def kernel(x, adj_s, adj_f, params):
    p = params
    n, nfeat = x.shape
    nh1 = p['s_W1'].shape[1]
    nh2 = p['s_W2'].shape[1]
    br_a = _row_block(n, 80)
    br_b = _row_block(n, 200)
    br_c = _row_block(n, 200)

    def vec2(v):
        return v.reshape(1, -1)

    xw1_s, xw1_f = pl.pallas_call(
        _pre_body,
        grid=(n // br_a,),
        in_specs=[_rows(br_a, nfeat), _full((nfeat, nh1)), _full((nfeat, nh1))],
        out_specs=[_rows(br_a, nh1), _rows(br_a, nh1)],
        out_shape=[jax.ShapeDtypeStruct((n, nh1), _F32)] * 2,
    )(x, p['s_W1'], p['f_W1'])

    t_s, t_f, m_s, m_f, s_s, s_f = pl.pallas_call(
        _passA_body,
        grid=(n // br_a,),
        in_specs=[_rows(br_a, n), _rows(br_a, n),
                  _full((n, nh1)), _full((n, nh1)),
                  _full((1, nh1)), _full((1, nh1)),
                  _full((nh1, nh2)), _full((nh1, nh2))],
        out_specs=[_rows(br_a, nh2), _rows(br_a, nh2),
                   _rows(br_a, n), _rows(br_a, n),
                   _rows(br_a, 1), _rows(br_a, 1)],
        out_shape=[jax.ShapeDtypeStruct((n, nh2), _F32),
                   jax.ShapeDtypeStruct((n, nh2), _F32),
                   jax.ShapeDtypeStruct((n, n), jnp.int8),
                   jax.ShapeDtypeStruct((n, n), jnp.int8),
                   jax.ShapeDtypeStruct((n, 1), _F32),
                   jax.ShapeDtypeStruct((n, 1), _F32)],
    )(adj_s, adj_f, xw1_s, xw1_f, vec2(p['s_b1']), vec2(p['f_b1']),
      p['s_W2'], p['f_W2'])

    hm, att2, hn, hd = pl.pallas_call(
        _passB_body,
        grid=(n // br_b,),
        in_specs=[_rows(br_b, n), _rows(br_b, n),
                  _rows(br_b, 1), _rows(br_b, 1),
                  _full((n, nh2)), _full((n, nh2)),
                  _full((1, nh2)), _full((1, nh2)),
                  _full((nh2, nh2)), _full((1, nh2)), _full((nh2, 1)),
                  _full((nh2, nh2)), _full((1, nh2)),
                  _full((nh2, nh1))],
        out_specs=[_rows(br_b, nh2), _rows(br_b, 2), _rows(br_b, nh2),
                   _rows(br_b, nh1)],
        out_shape=[jax.ShapeDtypeStruct((n, nh2), _F32),
                   jax.ShapeDtypeStruct((n, 2), _F32),
                   jax.ShapeDtypeStruct((n, nh2), _F32),
                   jax.ShapeDtypeStruct((n, nh1), _F32)],
    )(m_s, m_f, s_s, s_f, t_s, t_f, vec2(p['s_b2']), vec2(p['f_b2']),
      p['att_W'], vec2(p['att_b']), p['att_q'], p['mlp_W'], vec2(p['mlp_b']),
      p['dec_W1'])

    pi, disp, mean, recon = pl.pallas_call(
        _passC_body,
        grid=(n // br_c,),
        in_specs=[_rows(br_c, n), _rows(br_c, 1),
                  _full((n, nh1)),
                  _rows(br_c, nh2), _full((n, nh2)),
                  _full((1, nh1)),
                  _full((nh1, nfeat)), _full((1, nfeat)),
                  _full((nh1, nfeat)), _full((1, nfeat)),
                  _full((nh1, nfeat)), _full((1, nfeat))],
        out_specs=[_rows(br_c, nfeat), _rows(br_c, nfeat), _rows(br_c, nfeat),
                   _rows(br_c, n)],
        out_shape=[jax.ShapeDtypeStruct((n, nfeat), _F32),
                   jax.ShapeDtypeStruct((n, nfeat), _F32),
                   jax.ShapeDtypeStruct((n, nfeat), _F32),
                   jax.ShapeDtypeStruct((n, n), _F32)],
    )(m_s, s_s, hd, hn, hn, vec2(p['dec_b1']),
      p['dec_Wpi'], vec2(p['dec_bpi']),
      p['dec_Wdisp'], vec2(p['dec_bdisp']),
      p['dec_Wmean'], vec2(p['dec_bmean']))

    return (hm, recon, pi, disp, mean, att2.reshape(n, 2, 1))


# speedup vs baseline: 1.1792x; 1.1792x over previous
import jax
import jax.numpy as jnp
from jax import lax
from jax.experimental import pallas as pl

_F32 = jnp.float32
_F8 = jnp.float8_e4m3fn


def _row_block(n, target):
    for br in (target, 200, 80, 40, 8):
        if br <= n and n % br == 0:
            return br
    return n


def _full(shape):
    return pl.BlockSpec(shape, lambda i: (0,) * len(shape))


def _rows(br, ncols):
    return pl.BlockSpec((br, ncols), lambda i: (i, 0))


def _pre_body(x_ref, w1s_ref, w1f_ref, os_ref, of_ref):
    xb = x_ref[...]
    os_ref[...] = jnp.dot(xb, w1s_ref[...], preferred_element_type=_F32)
    of_ref[...] = jnp.dot(xb, w1f_ref[...], preferred_element_type=_F32)


def _passA_body(as_ref, af_ref, us_ref, uf_ref, b1s_ref, b1f_ref,
                w2s_ref, w2f_ref, ts_ref, tf_ref,
                ms_ref, mf_ref, ss_ref, sf_ref):
    a_s = as_ref[...]
    a_f = af_ref[...]
    hs = jnp.maximum(
        jnp.dot(a_s, us_ref[...], preferred_element_type=_F32)
        + b1s_ref[...], 0.0)
    ts_ref[...] = jnp.dot(hs, w2s_ref[...], preferred_element_type=_F32)
    hf = jnp.maximum(
        jnp.dot(a_f, uf_ref[...], preferred_element_type=_F32)
        + b1f_ref[...], 0.0)
    tf_ref[...] = jnp.dot(hf, w2f_ref[...], preferred_element_type=_F32)
    mx_s = jnp.max(a_s, axis=1, keepdims=True)
    mx_f = jnp.max(a_f, axis=1, keepdims=True)
    ss_ref[...] = mx_s
    sf_ref[...] = mx_f
    ms_ref[...] = (a_s * (1.0 / mx_s)).astype(_F8)
    mf_ref[...] = (a_f * (1.0 / mx_f)).astype(_F8)


def _passB_body(ms_ref, mf_ref, ss_ref, sf_ref, ts_ref, tf_ref,
                b2s_ref, b2f_ref, attW_ref, attb_ref, attq_ref,
                mlpW_ref, mlpb_ref, decW1_ref,
                hm_ref, att_ref, hn_ref, hd_ref):
    a_s_rec = ss_ref[...] * ms_ref[...].astype(_F32)
    a_f_rec = sf_ref[...] * mf_ref[...].astype(_F32)
    g_s = (jnp.dot(a_s_rec, ts_ref[...], preferred_element_type=_F32)
           + b2s_ref[...])
    g_f = (jnp.dot(a_f_rec, tf_ref[...], preferred_element_type=_F32)
           + b2f_ref[...])
    w_s = jnp.tanh(jnp.dot(g_s, attW_ref[...], preferred_element_type=_F32)
                   + attb_ref[...])
    w_f = jnp.tanh(jnp.dot(g_f, attW_ref[...], preferred_element_type=_F32)
                   + attb_ref[...])
    sc_s = jnp.dot(w_s, attq_ref[...], preferred_element_type=_F32)
    sc_f = jnp.dot(w_f, attq_ref[...], preferred_element_type=_F32)
    m = jnp.maximum(sc_s, sc_f)
    es = jnp.exp(sc_s - m)
    ef = jnp.exp(sc_f - m)
    den = es + ef
    a_s = es / den
    a_f = ef / den
    h = a_s * g_s + a_f * g_f
    hm = jnp.dot(h, mlpW_ref[...], preferred_element_type=_F32) + mlpb_ref[...]
    hm_ref[...] = hm
    att_ref[...] = jnp.concatenate([a_s, a_f], axis=1)
    nrm = jnp.sqrt(jnp.sum(hm * hm, axis=1, keepdims=True))
    hn_ref[...] = hm / (nrm + 1e-8)
    hd_ref[...] = jnp.dot(hm, decW1_ref[...], preferred_element_type=_F32)


def _passC_body(ms_ref, ss_ref, hd_ref, hnb_ref, hn_ref, db1_ref,
                wpi_ref, bpi_ref, wdisp_ref, bdisp_ref, wmean_ref, bmean_ref,
                pi_ref, disp_ref, mean_ref, recon_ref):
    a_s_rec = ss_ref[...] * ms_ref[...].astype(_F32)
    h1 = jnp.maximum(
        jnp.dot(a_s_rec, hd_ref[...], preferred_element_type=_F32)
        + db1_ref[...], 0.0)
    zpi = jnp.dot(h1, wpi_ref[...], preferred_element_type=_F32) + bpi_ref[...]
    pi_ref[...] = 1.0 / (1.0 + jnp.exp(-zpi))
    zd = jnp.dot(h1, wdisp_ref[...], preferred_element_type=_F32) + bdisp_ref[...]
    sp = jnp.maximum(zd, 0.0) + jnp.log1p(jnp.exp(-jnp.abs(zd)))
    disp_ref[...] = jnp.clip(sp, 1e-4, 1e4)
    zm = jnp.dot(h1, wmean_ref[...], preferred_element_type=_F32) + bmean_ref[...]
    mean_ref[...] = jnp.clip(jnp.exp(zm), 1e-5, 1e6)
    recon_ref[...] = lax.dot_general(
        hnb_ref[...], hn_ref[...], (((1,), (1,)), ((), ())),
        preferred_element_type=_F32)


def kernel(x, adj_s, adj_f, params):
    p = params
    n, nfeat = x.shape
    nh1 = p['s_W1'].shape[1]
    nh2 = p['s_W2'].shape[1]
    br_a = _row_block(n, 80)
    br_b = _row_block(n, 200)
    br_c = _row_block(n, 200)

    def vec2(v):
        return v.reshape(1, -1)

    xw1_s, xw1_f = pl.pallas_call(
        _pre_body,
        grid=(n // br_a,),
        in_specs=[_rows(br_a, nfeat), _full((nfeat, nh1)), _full((nfeat, nh1))],
        out_specs=[_rows(br_a, nh1), _rows(br_a, nh1)],
        out_shape=[jax.ShapeDtypeStruct((n, nh1), _F32)] * 2,
    )(x, p['s_W1'], p['f_W1'])

    t_s, t_f, m_s, m_f, s_s, s_f = pl.pallas_call(
        _passA_body,
        grid=(n // br_a,),
        in_specs=[_rows(br_a, n), _rows(br_a, n),
                  _full((n, nh1)), _full((n, nh1)),
                  _full((1, nh1)), _full((1, nh1)),
                  _full((nh1, nh2)), _full((nh1, nh2))],
        out_specs=[_rows(br_a, nh2), _rows(br_a, nh2),
                   _rows(br_a, n), _rows(br_a, n),
                   _rows(br_a, 1), _rows(br_a, 1)],
        out_shape=[jax.ShapeDtypeStruct((n, nh2), _F32),
                   jax.ShapeDtypeStruct((n, nh2), _F32),
                   jax.ShapeDtypeStruct((n, n), _F8),
                   jax.ShapeDtypeStruct((n, n), _F8),
                   jax.ShapeDtypeStruct((n, 1), _F32),
                   jax.ShapeDtypeStruct((n, 1), _F32)],
    )(adj_s, adj_f, xw1_s, xw1_f, vec2(p['s_b1']), vec2(p['f_b1']),
      p['s_W2'], p['f_W2'])

    hm, att2, hn, hd = pl.pallas_call(
        _passB_body,
        grid=(n // br_b,),
        in_specs=[_rows(br_b, n), _rows(br_b, n),
                  _rows(br_b, 1), _rows(br_b, 1),
                  _full((n, nh2)), _full((n, nh2)),
                  _full((1, nh2)), _full((1, nh2)),
                  _full((nh2, nh2)), _full((1, nh2)), _full((nh2, 1)),
                  _full((nh2, nh2)), _full((1, nh2)),
                  _full((nh2, nh1))],
        out_specs=[_rows(br_b, nh2), _rows(br_b, 2), _rows(br_b, nh2),
                   _rows(br_b, nh1)],
        out_shape=[jax.ShapeDtypeStruct((n, nh2), _F32),
                   jax.ShapeDtypeStruct((n, 2), _F32),
                   jax.ShapeDtypeStruct((n, nh2), _F32),
                   jax.ShapeDtypeStruct((n, nh1), _F32)],
    )(m_s, m_f, s_s, s_f, t_s, t_f, vec2(p['s_b2']), vec2(p['f_b2']),
      p['att_W'], vec2(p['att_b']), p['att_q'], p['mlp_W'], vec2(p['mlp_b']),
      p['dec_W1'])

    pi, disp, mean, recon = pl.pallas_call(
        _passC_body,
        grid=(n // br_c,),
        in_specs=[_rows(br_c, n), _rows(br_c, 1),
                  _full((n, nh1)),
                  _rows(br_c, nh2), _full((n, nh2)),
                  _full((1, nh1)),
                  _full((nh1, nfeat)), _full((1, nfeat)),
                  _full((nh1, nfeat)), _full((1, nfeat)),
                  _full((nh1, nfeat)), _full((1, nfeat))],
        out_specs=[_rows(br_c, nfeat), _rows(br_c, nfeat), _rows(br_c, nfeat),
                   _rows(br_c, n)],
        out_shape=[jax.ShapeDtypeStruct((n, nfeat), _F32),
                   jax.ShapeDtypeStruct((n, nfeat), _F32),
                   jax.ShapeDtypeStruct((n, nfeat), _F32),
                   jax.ShapeDtypeStruct((n, n), _F32)],
    )(m_s, s_s, hd, hn, hn, vec2(p['dec_b1']),
      p['dec_Wpi'], vec2(p['dec_bpi']),
      p['dec_Wdisp'], vec2(p['dec_bdisp']),
      p['dec_Wmean'], vec2(p['dec_bmean']))

    return (hm, recon, pi, disp, mean, att2.reshape(n, 2, 1))


# speedup vs baseline: 1.3230x; 1.1220x over previous
import jax
import jax.numpy as jnp
from jax import lax
from jax.experimental import pallas as pl

_F32 = jnp.float32
_F8 = jnp.float8_e4m3fn
_BF16 = jnp.bfloat16


def _mdot(m_bf16, v_f32):
    return lax.dot_general(m_bf16, v_f32, (((1,), (0,)), ((), ())),
                           preferred_element_type=_F32)


def _row_block(n, target):
    for br in (target, 200, 80, 40, 8):
        if br <= n and n % br == 0:
            return br
    return n


def _full(shape):
    return pl.BlockSpec(shape, lambda i: (0,) * len(shape))


def _rows(br, ncols):
    return pl.BlockSpec((br, ncols), lambda i: (i, 0))


def _pre_body(x_ref, w1s_ref, w1f_ref, os_ref, of_ref):
    xb = x_ref[...]
    os_ref[...] = jnp.dot(xb, w1s_ref[...], preferred_element_type=_F32)
    of_ref[...] = jnp.dot(xb, w1f_ref[...], preferred_element_type=_F32)


def _passA_body(as_ref, af_ref, us_ref, uf_ref, b1s_ref, b1f_ref,
                w2s_ref, w2f_ref, ts_ref, tf_ref,
                ms_ref, mf_ref, ss_ref, sf_ref):
    a_s = as_ref[...]
    a_f = af_ref[...]
    mx_s = jnp.max(a_s, axis=1, keepdims=True)
    mx_f = jnp.max(a_f, axis=1, keepdims=True)
    ss_ref[...] = mx_s
    sf_ref[...] = mx_f
    mq_s = (a_s * (1.0 / mx_s)).astype(_F8)
    mq_f = (a_f * (1.0 / mx_f)).astype(_F8)
    ms_ref[...] = mq_s
    mf_ref[...] = mq_f
    hs = jnp.maximum(
        mx_s * _mdot(mq_s, us_ref[...]) + b1s_ref[...], 0.0)
    ts_ref[...] = jnp.dot(hs, w2s_ref[...], preferred_element_type=_F32)
    hf = jnp.maximum(
        mx_f * _mdot(mq_f, uf_ref[...]) + b1f_ref[...], 0.0)
    tf_ref[...] = jnp.dot(hf, w2f_ref[...], preferred_element_type=_F32)


def _passB_body(ms_ref, mf_ref, ss_ref, sf_ref, ts_ref, tf_ref,
                b2s_ref, b2f_ref, attW_ref, attb_ref, attq_ref,
                mlpW_ref, mlpb_ref, decW1_ref,
                hm_ref, att_ref, hn_ref, hd_ref):
    g_s = (ss_ref[...] * _mdot(ms_ref[...], ts_ref[...])
           + b2s_ref[...])
    g_f = (sf_ref[...] * _mdot(mf_ref[...], tf_ref[...])
           + b2f_ref[...])
    w_s = jnp.tanh(jnp.dot(g_s, attW_ref[...], preferred_element_type=_F32)
                   + attb_ref[...])
    w_f = jnp.tanh(jnp.dot(g_f, attW_ref[...], preferred_element_type=_F32)
                   + attb_ref[...])
    sc_s = jnp.dot(w_s, attq_ref[...], preferred_element_type=_F32)
    sc_f = jnp.dot(w_f, attq_ref[...], preferred_element_type=_F32)
    m = jnp.maximum(sc_s, sc_f)
    es = jnp.exp(sc_s - m)
    ef = jnp.exp(sc_f - m)
    den = es + ef
    a_s = es / den
    a_f = ef / den
    h = a_s * g_s + a_f * g_f
    hm = jnp.dot(h, mlpW_ref[...], preferred_element_type=_F32) + mlpb_ref[...]
    hm_ref[...] = hm
    att_ref[...] = jnp.concatenate([a_s, a_f], axis=1)
    nrm = jnp.sqrt(jnp.sum(hm * hm, axis=1, keepdims=True))
    hn_ref[...] = hm / (nrm + 1e-8)
    hd_ref[...] = jnp.dot(hm, decW1_ref[...], preferred_element_type=_F32)


def _passC_body(ms_ref, ss_ref, hd_ref, hnb_ref, hn_ref, db1_ref,
                wpi_ref, bpi_ref, wdisp_ref, bdisp_ref, wmean_ref, bmean_ref,
                pi_ref, disp_ref, mean_ref, recon_ref):
    h1 = jnp.maximum(
        ss_ref[...] * _mdot(ms_ref[...], hd_ref[...])
        + db1_ref[...], 0.0)
    zpi = jnp.dot(h1, wpi_ref[...], preferred_element_type=_F32) + bpi_ref[...]
    pi_ref[...] = 1.0 / (1.0 + jnp.exp(-zpi))
    zd = jnp.dot(h1, wdisp_ref[...], preferred_element_type=_F32) + bdisp_ref[...]
    sp = jnp.maximum(zd, 0.0) + jnp.log1p(jnp.exp(-jnp.abs(zd)))
    disp_ref[...] = jnp.clip(sp, 1e-4, 1e4)
    zm = jnp.dot(h1, wmean_ref[...], preferred_element_type=_F32) + bmean_ref[...]
    mean_ref[...] = jnp.clip(jnp.exp(zm), 1e-5, 1e6)
    recon_ref[...] = lax.dot_general(
        hnb_ref[...], hn_ref[...], (((1,), (1,)), ((), ())),
        preferred_element_type=_F32)


def kernel(x, adj_s, adj_f, params):
    p = params
    n, nfeat = x.shape
    nh1 = p['s_W1'].shape[1]
    nh2 = p['s_W2'].shape[1]
    br_a = _row_block(n, 80)
    br_b = _row_block(n, 200)
    br_c = _row_block(n, 200)

    def vec2(v):
        return v.reshape(1, -1)

    xw1_s, xw1_f = pl.pallas_call(
        _pre_body,
        grid=(n // br_a,),
        in_specs=[_rows(br_a, nfeat), _full((nfeat, nh1)), _full((nfeat, nh1))],
        out_specs=[_rows(br_a, nh1), _rows(br_a, nh1)],
        out_shape=[jax.ShapeDtypeStruct((n, nh1), _F32)] * 2,
    )(x, p['s_W1'], p['f_W1'])

    t_s, t_f, m_s, m_f, s_s, s_f = pl.pallas_call(
        _passA_body,
        grid=(n // br_a,),
        in_specs=[_rows(br_a, n), _rows(br_a, n),
                  _full((n, nh1)), _full((n, nh1)),
                  _full((1, nh1)), _full((1, nh1)),
                  _full((nh1, nh2)), _full((nh1, nh2))],
        out_specs=[_rows(br_a, nh2), _rows(br_a, nh2),
                   _rows(br_a, n), _rows(br_a, n),
                   _rows(br_a, 1), _rows(br_a, 1)],
        out_shape=[jax.ShapeDtypeStruct((n, nh2), _F32),
                   jax.ShapeDtypeStruct((n, nh2), _F32),
                   jax.ShapeDtypeStruct((n, n), _F8),
                   jax.ShapeDtypeStruct((n, n), _F8),
                   jax.ShapeDtypeStruct((n, 1), _F32),
                   jax.ShapeDtypeStruct((n, 1), _F32)],
    )(adj_s, adj_f, xw1_s, xw1_f, vec2(p['s_b1']), vec2(p['f_b1']),
      p['s_W2'], p['f_W2'])

    hm, att2, hn, hd = pl.pallas_call(
        _passB_body,
        grid=(n // br_b,),
        in_specs=[_rows(br_b, n), _rows(br_b, n),
                  _rows(br_b, 1), _rows(br_b, 1),
                  _full((n, nh2)), _full((n, nh2)),
                  _full((1, nh2)), _full((1, nh2)),
                  _full((nh2, nh2)), _full((1, nh2)), _full((nh2, 1)),
                  _full((nh2, nh2)), _full((1, nh2)),
                  _full((nh2, nh1))],
        out_specs=[_rows(br_b, nh2), _rows(br_b, 2), _rows(br_b, nh2),
                   _rows(br_b, nh1)],
        out_shape=[jax.ShapeDtypeStruct((n, nh2), _F32),
                   jax.ShapeDtypeStruct((n, 2), _F32),
                   jax.ShapeDtypeStruct((n, nh2), _F32),
                   jax.ShapeDtypeStruct((n, nh1), _F32)],
    )(m_s, m_f, s_s, s_f, t_s, t_f, vec2(p['s_b2']), vec2(p['f_b2']),
      p['att_W'], vec2(p['att_b']), p['att_q'], p['mlp_W'], vec2(p['mlp_b']),
      p['dec_W1'])

    pi, disp, mean, recon = pl.pallas_call(
        _passC_body,
        grid=(n // br_c,),
        in_specs=[_rows(br_c, n), _rows(br_c, 1),
                  _full((n, nh1)),
                  _rows(br_c, nh2), _full((n, nh2)),
                  _full((1, nh1)),
                  _full((nh1, nfeat)), _full((1, nfeat)),
                  _full((nh1, nfeat)), _full((1, nfeat)),
                  _full((nh1, nfeat)), _full((1, nfeat))],
        out_specs=[_rows(br_c, nfeat), _rows(br_c, nfeat), _rows(br_c, nfeat),
                   _rows(br_c, n)],
        out_shape=[jax.ShapeDtypeStruct((n, nfeat), _F32),
                   jax.ShapeDtypeStruct((n, nfeat), _F32),
                   jax.ShapeDtypeStruct((n, nfeat), _F32),
                   jax.ShapeDtypeStruct((n, n), _F32)],
    )(m_s, s_s, hd, hn, hn, vec2(p['dec_b1']),
      p['dec_Wpi'], vec2(p['dec_bpi']),
      p['dec_Wdisp'], vec2(p['dec_bdisp']),
      p['dec_Wmean'], vec2(p['dec_bmean']))

    return (hm, recon, pi, disp, mean, att2.reshape(n, 2, 1))


# speedup vs baseline: 1.3763x; 1.0403x over previous
import jax
import jax.numpy as jnp
from jax import lax
from jax.experimental import pallas as pl

_F32 = jnp.float32
_F8 = jnp.float8_e4m3fn
_BF16 = jnp.bfloat16


def _mdot(m_bf16, v_f32):
    return lax.dot_general(m_bf16, v_f32, (((1,), (0,)), ((), ())),
                           preferred_element_type=_F32)


def _row_block(n, target):
    for br in (target, 200, 80, 40, 8):
        if br <= n and n % br == 0:
            return br
    return n


def _full(shape):
    return pl.BlockSpec(shape, lambda i: (0,) * len(shape))


def _rows(br, ncols):
    return pl.BlockSpec((br, ncols), lambda i: (i, 0))


def _pre_body(x_ref, w1s_ref, w1f_ref, os_ref, of_ref):
    xb = x_ref[...]
    os_ref[...] = jnp.dot(xb, w1s_ref[...], preferred_element_type=_F32)
    of_ref[...] = jnp.dot(xb, w1f_ref[...], preferred_element_type=_F32)


def _passA_body(a_ref, u_ref, b1_ref, w2_ref, t_ref, m_ref, s_ref):
    a = a_ref[...]
    mx = jnp.max(a, axis=1, keepdims=True)
    s_ref[...] = mx
    mq = (a * (1.0 / mx)).astype(_F8)
    m_ref[...] = mq
    h = jnp.maximum(mx * _mdot(mq, u_ref[...]) + b1_ref[...], 0.0)
    t_ref[...] = jnp.dot(h, w2_ref[...], preferred_element_type=_F32)


def _passB_body(ms_ref, mf_ref, ss_ref, sf_ref, ts_ref, tf_ref,
                b2s_ref, b2f_ref, attW_ref, attb_ref, attq_ref,
                mlpW_ref, mlpb_ref, decW1_ref,
                hm_ref, att_ref, hn_ref, hd_ref):
    g_s = (ss_ref[...] * _mdot(ms_ref[...], ts_ref[...])
           + b2s_ref[...])
    g_f = (sf_ref[...] * _mdot(mf_ref[...], tf_ref[...])
           + b2f_ref[...])
    w_s = jnp.tanh(jnp.dot(g_s, attW_ref[...], preferred_element_type=_F32)
                   + attb_ref[...])
    w_f = jnp.tanh(jnp.dot(g_f, attW_ref[...], preferred_element_type=_F32)
                   + attb_ref[...])
    sc_s = jnp.dot(w_s, attq_ref[...], preferred_element_type=_F32)
    sc_f = jnp.dot(w_f, attq_ref[...], preferred_element_type=_F32)
    m = jnp.maximum(sc_s, sc_f)
    es = jnp.exp(sc_s - m)
    ef = jnp.exp(sc_f - m)
    den = es + ef
    a_s = es / den
    a_f = ef / den
    h = a_s * g_s + a_f * g_f
    hm = jnp.dot(h, mlpW_ref[...], preferred_element_type=_F32) + mlpb_ref[...]
    hm_ref[...] = hm
    att_ref[...] = jnp.concatenate([a_s, a_f], axis=1)
    nrm = jnp.sqrt(jnp.sum(hm * hm, axis=1, keepdims=True))
    hn_ref[...] = hm / (nrm + 1e-8)
    hd_ref[...] = jnp.dot(hm, decW1_ref[...], preferred_element_type=_F32)


def _passC_body(ms_ref, ss_ref, hd_ref, hnb_ref, hn_ref, db1_ref,
                wpi_ref, bpi_ref, wdisp_ref, bdisp_ref, wmean_ref, bmean_ref,
                pi_ref, disp_ref, mean_ref, recon_ref):
    h1 = jnp.maximum(
        ss_ref[...] * _mdot(ms_ref[...], hd_ref[...])
        + db1_ref[...], 0.0)
    zpi = jnp.dot(h1, wpi_ref[...], preferred_element_type=_F32) + bpi_ref[...]
    pi_ref[...] = 1.0 / (1.0 + jnp.exp(-zpi))
    zd = jnp.dot(h1, wdisp_ref[...], preferred_element_type=_F32) + bdisp_ref[...]
    sp = jnp.maximum(zd, 0.0) + jnp.log1p(jnp.exp(-jnp.abs(zd)))
    disp_ref[...] = jnp.clip(sp, 1e-4, 1e4)
    zm = jnp.dot(h1, wmean_ref[...], preferred_element_type=_F32) + bmean_ref[...]
    mean_ref[...] = jnp.clip(jnp.exp(zm), 1e-5, 1e6)
    recon_ref[...] = lax.dot_general(
        hnb_ref[...], hn_ref[...], (((1,), (1,)), ((), ())),
        preferred_element_type=_F32)


def kernel(x, adj_s, adj_f, params):
    p = params
    n, nfeat = x.shape
    nh1 = p['s_W1'].shape[1]
    nh2 = p['s_W2'].shape[1]
    br_a = _row_block(n, 200)
    br_b = _row_block(n, 400)
    br_c = _row_block(n, 400)

    def vec2(v):
        return v.reshape(1, -1)

    xw1_s, xw1_f = pl.pallas_call(
        _pre_body,
        grid=(n // br_a,),
        in_specs=[_rows(br_a, nfeat), _full((nfeat, nh1)), _full((nfeat, nh1))],
        out_specs=[_rows(br_a, nh1), _rows(br_a, nh1)],
        out_shape=[jax.ShapeDtypeStruct((n, nh1), _F32)] * 2,
    )(x, p['s_W1'], p['f_W1'])

    def stage_a(adj, xw1, b1, w2):
        return pl.pallas_call(
            _passA_body,
            grid=(n // br_a,),
            in_specs=[_rows(br_a, n), _full((n, nh1)),
                      _full((1, nh1)), _full((nh1, nh2))],
            out_specs=[_rows(br_a, nh2), _rows(br_a, n), _rows(br_a, 1)],
            out_shape=[jax.ShapeDtypeStruct((n, nh2), _F32),
                       jax.ShapeDtypeStruct((n, n), _F8),
                       jax.ShapeDtypeStruct((n, 1), _F32)],
        )(adj, xw1, vec2(b1), w2)

    t_s, m_s, s_s = stage_a(adj_s, xw1_s, p['s_b1'], p['s_W2'])
    t_f, m_f, s_f = stage_a(adj_f, xw1_f, p['f_b1'], p['f_W2'])

    hm, att2, hn, hd = pl.pallas_call(
        _passB_body,
        grid=(n // br_b,),
        in_specs=[_rows(br_b, n), _rows(br_b, n),
                  _rows(br_b, 1), _rows(br_b, 1),
                  _full((n, nh2)), _full((n, nh2)),
                  _full((1, nh2)), _full((1, nh2)),
                  _full((nh2, nh2)), _full((1, nh2)), _full((nh2, 1)),
                  _full((nh2, nh2)), _full((1, nh2)),
                  _full((nh2, nh1))],
        out_specs=[_rows(br_b, nh2), _rows(br_b, 2), _rows(br_b, nh2),
                   _rows(br_b, nh1)],
        out_shape=[jax.ShapeDtypeStruct((n, nh2), _F32),
                   jax.ShapeDtypeStruct((n, 2), _F32),
                   jax.ShapeDtypeStruct((n, nh2), _F32),
                   jax.ShapeDtypeStruct((n, nh1), _F32)],
    )(m_s, m_f, s_s, s_f, t_s, t_f, vec2(p['s_b2']), vec2(p['f_b2']),
      p['att_W'], vec2(p['att_b']), p['att_q'], p['mlp_W'], vec2(p['mlp_b']),
      p['dec_W1'])

    pi, disp, mean, recon = pl.pallas_call(
        _passC_body,
        grid=(n // br_c,),
        in_specs=[_rows(br_c, n), _rows(br_c, 1),
                  _full((n, nh1)),
                  _rows(br_c, nh2), _full((n, nh2)),
                  _full((1, nh1)),
                  _full((nh1, nfeat)), _full((1, nfeat)),
                  _full((nh1, nfeat)), _full((1, nfeat)),
                  _full((nh1, nfeat)), _full((1, nfeat))],
        out_specs=[_rows(br_c, nfeat), _rows(br_c, nfeat), _rows(br_c, nfeat),
                   _rows(br_c, n)],
        out_shape=[jax.ShapeDtypeStruct((n, nfeat), _F32),
                   jax.ShapeDtypeStruct((n, nfeat), _F32),
                   jax.ShapeDtypeStruct((n, nfeat), _F32),
                   jax.ShapeDtypeStruct((n, n), _F32)],
    )(m_s, s_s, hd, hn, hn, vec2(p['dec_b1']),
      p['dec_Wpi'], vec2(p['dec_bpi']),
      p['dec_Wdisp'], vec2(p['dec_bdisp']),
      p['dec_Wmean'], vec2(p['dec_bmean']))

    return (hm, recon, pi, disp, mean, att2.reshape(n, 2, 1))


# speedup vs baseline: 1.3786x; 1.0017x over previous
import jax
import jax.numpy as jnp
from jax import lax
from jax.experimental import pallas as pl

_F32 = jnp.float32
_F8 = jnp.float8_e4m3fn
_BF16 = jnp.bfloat16


def _mdot(m_bf16, v_f32):
    return lax.dot_general(m_bf16, v_f32, (((1,), (0,)), ((), ())),
                           preferred_element_type=_F32)


def _row_block(n, target):
    for br in (target, 200, 80, 40, 8):
        if br <= n and n % br == 0:
            return br
    return n


def _full(shape):
    return pl.BlockSpec(shape, lambda i: (0,) * len(shape))


def _rows(br, ncols):
    return pl.BlockSpec((br, ncols), lambda i: (i, 0))


def _pre_body(x_ref, w1s_ref, w1f_ref, os_ref, of_ref):
    xb = x_ref[...]
    os_ref[...] = jnp.dot(xb, w1s_ref[...], preferred_element_type=_F32)
    of_ref[...] = jnp.dot(xb, w1f_ref[...], preferred_element_type=_F32)


def _passA_body(a_ref, u_ref, b1_ref, w2_ref, t_ref, m_ref, s_ref):
    a = a_ref[...]
    mx = jnp.max(a, axis=1, keepdims=True)
    s_ref[...] = mx
    mq = (a * (1.0 / mx)).astype(_F8)
    m_ref[...] = mq
    h = jnp.maximum(mx * _mdot(mq, u_ref[...]) + b1_ref[...], 0.0)
    t_ref[...] = jnp.dot(h, w2_ref[...], preferred_element_type=_F32)


def _passB_body(ms_ref, mf_ref, ss_ref, sf_ref, ts_ref, tf_ref,
                b2s_ref, b2f_ref, attW_ref, attb_ref, attq_ref,
                mlpW_ref, mlpb_ref, decW1_ref,
                hm_ref, att_ref, hn_ref, hd_ref):
    g_s = (ss_ref[...] * _mdot(ms_ref[...], ts_ref[...])
           + b2s_ref[...])
    g_f = (sf_ref[...] * _mdot(mf_ref[...], tf_ref[...])
           + b2f_ref[...])
    w_s = jnp.tanh(jnp.dot(g_s, attW_ref[...], preferred_element_type=_F32)
                   + attb_ref[...])
    w_f = jnp.tanh(jnp.dot(g_f, attW_ref[...], preferred_element_type=_F32)
                   + attb_ref[...])
    sc_s = jnp.dot(w_s, attq_ref[...], preferred_element_type=_F32)
    sc_f = jnp.dot(w_f, attq_ref[...], preferred_element_type=_F32)
    m = jnp.maximum(sc_s, sc_f)
    es = jnp.exp(sc_s - m)
    ef = jnp.exp(sc_f - m)
    den = es + ef
    a_s = es / den
    a_f = ef / den
    h = a_s * g_s + a_f * g_f
    hm = jnp.dot(h, mlpW_ref[...], preferred_element_type=_F32) + mlpb_ref[...]
    hm_ref[...] = hm
    att_ref[...] = jnp.concatenate([a_s, a_f], axis=1)
    nrm = jnp.sqrt(jnp.sum(hm * hm, axis=1, keepdims=True))
    hn_ref[...] = hm / (nrm + 1e-8)
    hd_ref[...] = jnp.dot(hm, decW1_ref[...], preferred_element_type=_F32)


def _passC_body(ms_ref, ss_ref, hd_ref, hnb_ref, hn_ref, db1_ref,
                wpi_ref, bpi_ref, wdisp_ref, bdisp_ref, wmean_ref, bmean_ref,
                pi_ref, disp_ref, mean_ref, recon_ref):
    h1 = jnp.maximum(
        ss_ref[...] * _mdot(ms_ref[...], hd_ref[...])
        + db1_ref[...], 0.0)
    zpi = jnp.dot(h1, wpi_ref[...], preferred_element_type=_F32) + bpi_ref[...]
    pi_ref[...] = 1.0 / (1.0 + jnp.exp(-zpi))
    zd = jnp.dot(h1, wdisp_ref[...], preferred_element_type=_F32) + bdisp_ref[...]
    sp = jnp.maximum(zd, 0.0) + jnp.log1p(jnp.exp(-jnp.abs(zd)))
    disp_ref[...] = jnp.clip(sp, 1e-4, 1e4)
    zm = jnp.dot(h1, wmean_ref[...], preferred_element_type=_F32) + bmean_ref[...]
    mean_ref[...] = jnp.clip(jnp.exp(zm), 1e-5, 1e6)
    recon_ref[...] = lax.dot_general(
        hnb_ref[...].astype(_BF16), hn_ref[...], (((1,), (1,)), ((), ())),
        preferred_element_type=_F32)


def kernel(x, adj_s, adj_f, params):
    p = params
    n, nfeat = x.shape
    nh1 = p['s_W1'].shape[1]
    nh2 = p['s_W2'].shape[1]
    br_a = _row_block(n, 200)
    br_b = _row_block(n, 400)
    br_c = _row_block(n, 400)

    def vec2(v):
        return v.reshape(1, -1)

    xw1_s, xw1_f = pl.pallas_call(
        _pre_body,
        grid=(n // br_a,),
        in_specs=[_rows(br_a, nfeat), _full((nfeat, nh1)), _full((nfeat, nh1))],
        out_specs=[_rows(br_a, nh1), _rows(br_a, nh1)],
        out_shape=[jax.ShapeDtypeStruct((n, nh1), _F32)] * 2,
    )(x, p['s_W1'], p['f_W1'])

    def stage_a(adj, xw1, b1, w2):
        return pl.pallas_call(
            _passA_body,
            grid=(n // br_a,),
            in_specs=[_rows(br_a, n), _full((n, nh1)),
                      _full((1, nh1)), _full((nh1, nh2))],
            out_specs=[_rows(br_a, nh2), _rows(br_a, n), _rows(br_a, 1)],
            out_shape=[jax.ShapeDtypeStruct((n, nh2), _F32),
                       jax.ShapeDtypeStruct((n, n), _F8),
                       jax.ShapeDtypeStruct((n, 1), _F32)],
        )(adj, xw1, vec2(b1), w2)

    t_s, m_s, s_s = stage_a(adj_s, xw1_s, p['s_b1'], p['s_W2'])
    t_f, m_f, s_f = stage_a(adj_f, xw1_f, p['f_b1'], p['f_W2'])

    hm, att2, hn, hd = pl.pallas_call(
        _passB_body,
        grid=(n // br_b,),
        in_specs=[_rows(br_b, n), _rows(br_b, n),
                  _rows(br_b, 1), _rows(br_b, 1),
                  _full((n, nh2)), _full((n, nh2)),
                  _full((1, nh2)), _full((1, nh2)),
                  _full((nh2, nh2)), _full((1, nh2)), _full((nh2, 1)),
                  _full((nh2, nh2)), _full((1, nh2)),
                  _full((nh2, nh1))],
        out_specs=[_rows(br_b, nh2), _rows(br_b, 2), _rows(br_b, nh2),
                   _rows(br_b, nh1)],
        out_shape=[jax.ShapeDtypeStruct((n, nh2), _F32),
                   jax.ShapeDtypeStruct((n, 2), _F32),
                   jax.ShapeDtypeStruct((n, nh2), _F32),
                   jax.ShapeDtypeStruct((n, nh1), _F32)],
    )(m_s, m_f, s_s, s_f, t_s, t_f, vec2(p['s_b2']), vec2(p['f_b2']),
      p['att_W'], vec2(p['att_b']), p['att_q'], p['mlp_W'], vec2(p['mlp_b']),
      p['dec_W1'])

    pi, disp, mean, recon = pl.pallas_call(
        _passC_body,
        grid=(n // br_c,),
        in_specs=[_rows(br_c, n), _rows(br_c, 1),
                  _full((n, nh1)),
                  _rows(br_c, nh2), _full((n, nh2)),
                  _full((1, nh1)),
                  _full((nh1, nfeat)), _full((1, nfeat)),
                  _full((nh1, nfeat)), _full((1, nfeat)),
                  _full((nh1, nfeat)), _full((1, nfeat))],
        out_specs=[_rows(br_c, nfeat), _rows(br_c, nfeat), _rows(br_c, nfeat),
                   _rows(br_c, n)],
        out_shape=[jax.ShapeDtypeStruct((n, nfeat), _F32),
                   jax.ShapeDtypeStruct((n, nfeat), _F32),
                   jax.ShapeDtypeStruct((n, nfeat), _F32),
                   jax.ShapeDtypeStruct((n, n), _F32)],
    )(m_s, s_s, hd, hn, hn, vec2(p['dec_b1']),
      p['dec_Wpi'], vec2(p['dec_bpi']),
      p['dec_Wdisp'], vec2(p['dec_bdisp']),
      p['dec_Wmean'], vec2(p['dec_bmean']))

    return (hm, recon, pi, disp, mean, att2.reshape(n, 2, 1))
